# Initial kernel scaffold; baseline (speedup 1.0000x reference)
import jax
import jax.numpy as jnp
from jax.experimental import pallas as pl


def kernel(v_1, v_2, adj_indices, adj_weights, laplace_w):
    raise NotImplementedError("write your pallas kernel here")


# speedup vs baseline: 17.9157x; 17.9157x over previous
import functools

import jax
import jax.numpy as jnp
from jax import lax
from jax.experimental import pallas as pl
from jax.experimental.pallas import tpu as pltpu
from jax.experimental.pallas import tpu_sc as plsc

_N = 27554
_K = 9
_NC = 2
_NS = 16
_NW = _NC * _NS
_VPW = 864
_NPAD = _NW * _VPW
_G = _VPW // 16
_FLAT = 3 * _NPAD


def _dv_body(a_ref, b_ref, o_ref):
    o_ref[...] = a_ref[...] - b_ref[...]


def _dv_table(v1f, v2f):
    return pl.pallas_call(
        _dv_body,
        out_shape=jax.ShapeDtypeStruct((_FLAT // 128, 128), jnp.float32),
    )(v1f, v2f)


def _sc_body(dv_hbm, idx_hbm, w_hbm, lw_hbm, out_hbm, dv_v, idx_v, w_v, lw_v, tot_v):
    cid = lax.axis_index("c")
    sid = lax.axis_index("s")
    wid = sid * _NC + cid
    base = wid * _VPW
    pltpu.sync_copy(dv_hbm, dv_v)
    pltpu.sync_copy(idx_hbm.at[wid], idx_v)
    pltpu.sync_copy(w_hbm.at[pl.ds(base, _VPW)], w_v)
    pltpu.sync_copy(lw_hbm.at[pl.ds(base, _VPW)], lw_v)
    lane3 = lax.iota(jnp.int32, 16) * 3

    def body(g, tot):
        s0 = g * 16
        accx = jnp.zeros((16,), jnp.float32)
        accy = jnp.zeros((16,), jnp.float32)
        accz = jnp.zeros((16,), jnp.float32)
        for k in range(_K):
            i3 = idx_v[k, pl.ds(s0, 16)] * 3
            accx = accx + plsc.load_gather(dv_v, [i3])
            accy = accy + plsc.load_gather(dv_v, [i3 + 1])
            accz = accz + plsc.load_gather(dv_v, [i3 + 2])
        own = (base + s0) * 3 + lane3
        ox = plsc.load_gather(dv_v, [own])
        oy = plsc.load_gather(dv_v, [own + 1])
        oz = plsc.load_gather(dv_v, [own + 2])
        rcp = 1.0 / w_v[pl.ds(s0, 16)]
        tx = ox - accx * rcp
        ty = oy - accy * rcp
        tz = oz - accz * rcp
        return tot + lw_v[pl.ds(s0, 16)] * (tx * tx + ty * ty + tz * tz)

    tot = lax.fori_loop(0, _G, body, jnp.zeros((16,), jnp.float32))
    tot_v[...] = tot
    pltpu.sync_copy(tot_v, out_hbm.at[wid])


_sc_partials = functools.partial(
    pl.kernel,
    out_type=jax.ShapeDtypeStruct((_NW, 16), jnp.float32),
    mesh=plsc.VectorSubcoreMesh(core_axis_name="c", subcore_axis_name="s"),
    scratch_types=[
        pltpu.VMEM((_FLAT,), jnp.float32),
        pltpu.VMEM((_K, _VPW), jnp.int32),
        pltpu.VMEM((_VPW,), jnp.float32),
        pltpu.VMEM((_VPW,), jnp.float32),
        pltpu.VMEM((16,), jnp.float32),
    ],
    compiler_params=pltpu.CompilerParams(needs_layout_passes=False),
)(_sc_body)


def kernel(v_1, v_2, adj_indices, adj_weights, laplace_w):
    v1f = jnp.pad(v_1.reshape(-1), (0, _FLAT - 3 * _N)).reshape(_FLAT // 128, 128)
    v2f = jnp.pad(v_2.reshape(-1), (0, _FLAT - 3 * _N)).reshape(_FLAT // 128, 128)
    dv = _dv_table(v1f, v2f).reshape(_FLAT)

    idx = adj_indices[:, :_K].astype(jnp.int32)
    idx = jnp.pad(idx, ((0, _NPAD - _N), (0, 0)))
    idx = idx.T.reshape(_K, _NW, _VPW).transpose(1, 0, 2)

    w = jnp.pad(adj_weights.reshape(-1), (0, _NPAD - _N), constant_values=1.0)
    lw = jnp.pad(laplace_w.reshape(-1), (0, _NPAD - _N))

    partials = _sc_partials(dv, idx, w, lw)
    return jnp.sum(partials) / (_N * 3)
